# baseline (speedup 1.0000x reference)
import functools

import jax
import jax.numpy as jnp
from jax.experimental import pallas as pl
from jax.experimental.pallas import tpu as pltpu
from jax.experimental.pallas import tpu_sc as plsc

N_TOTAL = 100000
BLOCK = 4000
SMALL_ROWS = 128
PATOM_ROWS = 1024
SC_WINDOW = 256
SC_PAD_N = 102400


def _sc_gather(table, flat_idx):
    n = flat_idx.shape[1]
    mesh = plsc.VectorSubcoreMesh(core_axis_name="core",
                                  subcore_axis_name="subcore")

    @pl.kernel(out_type=jax.ShapeDtypeStruct((n, table.shape[1]),
                                             table.dtype),
               mesh=mesh)
    def k(t_hbm, i_hbm, o_hbm):
        def body(i_vmem, o_vmem):
            pltpu.sync_copy(t_hbm.at[i_vmem.at[0]], o_vmem)

        pltpu.emit_pipeline(
            body,
            grid=(n // SC_WINDOW,),
            in_specs=[pl.BlockSpec((1, SC_WINDOW), lambda i: (0, i))],
            out_specs=[pl.BlockSpec((SC_WINDOW, table.shape[1]),
                                    lambda i: (i, 0))],
            core_axis_name=("core", "subcore"),
            dimension_semantics=(pltpu.PARALLEL,),
        )(i_hbm, o_hbm)

    return k(table, flat_idx)


def _fwd_kernel(idx_ref, fmapt_ref, s_ref, g_ref, wp_ref, w2_ref,
                b2_ref, o_ref):
    bf16 = jnp.bfloat16
    idx = idx_ref[0]
    idx_bf = idx.astype(bf16)

    sel_t = jax.lax.dot_general(
        fmapt_ref[...], idx_bf, (((1,), (0,)), ((), ())),
        preferred_element_type=jnp.float32)
    row_s = jax.lax.broadcasted_iota(jnp.int32, (SMALL_ROWS, 1), 0)
    mh_t = (sel_t == row_s.astype(jnp.float32)).astype(bf16)

    acc = jax.lax.dot_general(
        mh_t, s_ref[...], (((0,), (0,)), ((), ())),
        preferred_element_type=jnp.float32)

    g = g_ref[:, :32].astype(bf16)
    acc = acc + jnp.dot(g, wp_ref[...], preferred_element_type=jnp.float32)

    act = acc * jax.nn.sigmoid(acc)
    o_ref[...] = (
        jnp.dot(act.astype(bf16), w2_ref[...],
                preferred_element_type=jnp.float32)
        + b2_ref[...]
    )


@functools.partial(jax.jit, static_argnames=())
def kernel(node_element, node_charge, node_aromatic, node_hybridization,
           node_num_rings, node_is_donor, node_is_acceptor, node_is_positive,
           node_is_negative, node_is_hydrophobe, node_patom_token, node_type,
           node_pres_residue_type, node_patom_is_donor, node_patom_is_acceptor,
           node_patom_is_positive, node_patom_is_negative,
           node_patom_is_hydrophobic, elem_table, charge_W, charge_b,
           aromatic_table, hybrid_table, rings_table, bool_W, bool_b,
           patom_table, type_table, res_table, patom_bool_W, patom_bool_b,
           W1, b1, W2, b2):
    n = node_element.shape[0]
    nb = n // BLOCK

    f32 = jnp.float32

    p_elem = elem_table @ W1[0:32]
    p_charge = (jnp.arange(3, dtype=f32)[:, None] @ charge_W
                + charge_b[None, :]) @ W1[32:40]
    p_arom = aromatic_table @ W1[40:48]
    p_hyb = hybrid_table @ W1[48:64]
    p_rings = rings_table @ W1[64:72]
    bits = ((jnp.arange(32)[:, None] >> jnp.arange(5)[None, :]) & 1).astype(f32)
    p_bool = (bits @ bool_W + bool_b[None, :]) @ W1[72:88]
    p_type = type_table @ W1[120:136]
    p_res = res_table @ W1[136:152]
    p_pbool = (bits @ patom_bool_W + patom_bool_b[None, :]) @ W1[152:168]

    p_elem = p_elem + b1[None, :]

    small = jnp.concatenate(
        [p_elem, p_charge, p_arom, p_hyb, p_rings, p_bool, p_type, p_res,
         p_pbool], axis=0)
    small = jnp.pad(small, ((0, SMALL_ROWS - small.shape[0]), (0, 0)))
    small = small.astype(jnp.bfloat16)

    wp = W1[88:120].astype(jnp.bfloat16)
    pt128 = jnp.pad(patom_table, ((0, 0), (0, 96)))

    def cl(x, r):
        return jnp.clip(x, 0, r - 1).astype(jnp.int32)

    code_bool = (cl(node_is_donor, 2) + 2 * cl(node_is_acceptor, 2)
                 + 4 * cl(node_is_positive, 2) + 8 * cl(node_is_negative, 2)
                 + 16 * cl(node_is_hydrophobe, 2))
    code_pbool = (cl(node_patom_is_donor, 2) + 2 * cl(node_patom_is_acceptor, 2)
                  + 4 * cl(node_patom_is_positive, 2)
                  + 8 * cl(node_patom_is_negative, 2)
                  + 16 * cl(node_patom_is_hydrophobic, 2))

    sizes = [13, 3, 2, 7, 5, 32, 4, 22, 32]
    offs = [0, 13, 16, 18, 25, 30, 62, 66, 88]
    idx_list = [
        cl(node_element, 13) + 0,
        cl(node_charge, 3) + 13,
        cl(node_aromatic, 2) + 16,
        cl(node_hybridization, 7) + 18,
        cl(node_num_rings, 5) + 25,
        code_bool + 30,
        cl(node_type, 4) + 62,
        cl(node_pres_residue_type, 22) + 66,
        code_pbool + 88,
    ]
    pat_idx = cl(node_patom_token, 1000)
    idx = jnp.stack(idx_list + [jnp.zeros((n,), jnp.int32)] * 7, axis=0)
    idx = idx.reshape(16, nb, BLOCK).transpose(1, 0, 2)

    pat_pad = jnp.pad(pat_idx, (0, SC_PAD_N - n))
    g = _sc_gather(pt128, pat_pad.reshape(1, SC_PAD_N))

    fmapt = jnp.zeros((SMALL_ROWS, 16), jnp.float32)
    for k, (sz, off) in enumerate(zip(sizes, offs)):
        fmapt = fmapt.at[off:off + sz, k].set(1.0)
    fmapt = fmapt.astype(jnp.bfloat16)

    out = pl.pallas_call(
        _fwd_kernel,
        grid=(nb,),
        in_specs=[
            pl.BlockSpec((1, 16, BLOCK), lambda i: (i, 0, 0)),
            pl.BlockSpec((SMALL_ROWS, 16), lambda i: (0, 0)),
            pl.BlockSpec((SMALL_ROWS, 256), lambda i: (0, 0)),
            pl.BlockSpec((BLOCK, 128), lambda i: (i, 0)),
            pl.BlockSpec((32, 256), lambda i: (0, 0)),
            pl.BlockSpec((256, 256), lambda i: (0, 0)),
            pl.BlockSpec((1, 256), lambda i: (0, 0)),
        ],
        out_specs=pl.BlockSpec((BLOCK, 256), lambda i: (i, 0)),
        out_shape=jax.ShapeDtypeStruct((n, 256), f32),
    )(idx, fmapt, small, g, wp, W2.astype(jnp.bfloat16), b2[None, :])
    return out


# speedup vs baseline: 1.6223x; 1.6223x over previous
import functools

import jax
import jax.numpy as jnp
from jax.experimental import pallas as pl
from jax.experimental.pallas import tpu as pltpu
from jax.experimental.pallas import tpu_sc as plsc

N_TOTAL = 100000
BLOCK = 4000
SMALL_ROWS = 128
PATOM_ROWS = 1024
SC_WINDOW = 128
SC_N = 64000
NSC_BLOCKS = SC_N // BLOCK


def _sc_gather(table, flat_idx):
    n = flat_idx.shape[1]
    mesh = plsc.VectorSubcoreMesh(core_axis_name="core",
                                  subcore_axis_name="subcore")

    @pl.kernel(out_type=jax.ShapeDtypeStruct((n, table.shape[1]),
                                             table.dtype),
               mesh=mesh)
    def k(t_hbm, i_hbm, o_hbm):
        def body(i_vmem, o_vmem):
            pltpu.sync_copy(t_hbm.at[i_vmem.at[0]], o_vmem)

        pltpu.emit_pipeline(
            body,
            grid=(n // SC_WINDOW,),
            in_specs=[pl.BlockSpec((1, SC_WINDOW), lambda i: (0, i))],
            out_specs=[pl.BlockSpec((SC_WINDOW, table.shape[1]),
                                    lambda i: (i, 0))],
            core_axis_name=("core", "subcore"),
            dimension_semantics=(pltpu.PARALLEL,),
        )(i_hbm, o_hbm)

    return k(table, flat_idx)


def _fwd_kernel(idx_ref, fmapt_ref, s_ref, g_ref, pt_ref, wp_ref, w2_ref,
                b2_ref, o_ref, acc_ref):
    bf16 = jnp.bfloat16
    i = pl.program_id(0)
    idx = idx_ref[0]
    idx_bf = idx.astype(bf16)

    sel_t = jax.lax.dot_general(
        fmapt_ref[...], idx_bf, (((1,), (0,)), ((), ())),
        preferred_element_type=jnp.float32)
    row_s = jax.lax.broadcasted_iota(jnp.int32, (SMALL_ROWS, 1), 0)
    mh_t = (sel_t == row_s.astype(jnp.float32)).astype(bf16)

    acc_ref[...] = jax.lax.dot_general(
        mh_t, s_ref[...], (((0,), (0,)), ((), ())),
        preferred_element_type=jnp.float32)

    @pl.when(i < NSC_BLOCKS)
    def _sc_part():
        g = g_ref[:, :32].astype(bf16)
        acc_ref[...] += jnp.dot(g, wp_ref[...],
                                preferred_element_type=jnp.float32)

    @pl.when(i >= NSC_BLOCKS)
    def _tc_part():
        col_p = jax.lax.broadcasted_iota(jnp.int32, (1, PATOM_ROWS), 1)
        oh = (idx[9][:, None] == col_p).astype(bf16)
        gg = jnp.dot(oh, pt_ref[...], preferred_element_type=jnp.float32)
        acc_ref[...] += jnp.dot(gg.astype(bf16), wp_ref[...],
                                preferred_element_type=jnp.float32)

    acc = acc_ref[...]
    act = acc * jax.nn.sigmoid(acc)
    o_ref[...] = (
        jnp.dot(act.astype(bf16), w2_ref[...],
                preferred_element_type=jnp.float32)
        + b2_ref[...]
    )


@functools.partial(jax.jit, static_argnames=())
def kernel(node_element, node_charge, node_aromatic, node_hybridization,
           node_num_rings, node_is_donor, node_is_acceptor, node_is_positive,
           node_is_negative, node_is_hydrophobe, node_patom_token, node_type,
           node_pres_residue_type, node_patom_is_donor, node_patom_is_acceptor,
           node_patom_is_positive, node_patom_is_negative,
           node_patom_is_hydrophobic, elem_table, charge_W, charge_b,
           aromatic_table, hybrid_table, rings_table, bool_W, bool_b,
           patom_table, type_table, res_table, patom_bool_W, patom_bool_b,
           W1, b1, W2, b2):
    n = node_element.shape[0]
    nb = n // BLOCK

    f32 = jnp.float32

    p_elem = elem_table @ W1[0:32]
    p_charge = (jnp.arange(3, dtype=f32)[:, None] @ charge_W
                + charge_b[None, :]) @ W1[32:40]
    p_arom = aromatic_table @ W1[40:48]
    p_hyb = hybrid_table @ W1[48:64]
    p_rings = rings_table @ W1[64:72]
    bits = ((jnp.arange(32)[:, None] >> jnp.arange(5)[None, :]) & 1).astype(f32)
    p_bool = (bits @ bool_W + bool_b[None, :]) @ W1[72:88]
    p_type = type_table @ W1[120:136]
    p_res = res_table @ W1[136:152]
    p_pbool = (bits @ patom_bool_W + patom_bool_b[None, :]) @ W1[152:168]

    p_elem = p_elem + b1[None, :]

    small = jnp.concatenate(
        [p_elem, p_charge, p_arom, p_hyb, p_rings, p_bool, p_type, p_res,
         p_pbool], axis=0)
    small = jnp.pad(small, ((0, SMALL_ROWS - small.shape[0]), (0, 0)))
    small = small.astype(jnp.bfloat16)

    wp = W1[88:120].astype(jnp.bfloat16)
    pt128 = jnp.pad(patom_table, ((0, 0), (0, 96)))
    pt = jnp.pad(patom_table, ((0, PATOM_ROWS - patom_table.shape[0]), (0, 0)))
    pt = pt.astype(jnp.bfloat16)

    def cl(x, r):
        return jnp.clip(x, 0, r - 1).astype(jnp.int32)

    code_bool = (cl(node_is_donor, 2) + 2 * cl(node_is_acceptor, 2)
                 + 4 * cl(node_is_positive, 2) + 8 * cl(node_is_negative, 2)
                 + 16 * cl(node_is_hydrophobe, 2))
    code_pbool = (cl(node_patom_is_donor, 2) + 2 * cl(node_patom_is_acceptor, 2)
                  + 4 * cl(node_patom_is_positive, 2)
                  + 8 * cl(node_patom_is_negative, 2)
                  + 16 * cl(node_patom_is_hydrophobic, 2))

    sizes = [13, 3, 2, 7, 5, 32, 4, 22, 32]
    offs = [0, 13, 16, 18, 25, 30, 62, 66, 88]
    idx_list = [
        cl(node_element, 13) + 0,
        cl(node_charge, 3) + 13,
        cl(node_aromatic, 2) + 16,
        cl(node_hybridization, 7) + 18,
        cl(node_num_rings, 5) + 25,
        code_bool + 30,
        cl(node_type, 4) + 62,
        cl(node_pres_residue_type, 22) + 66,
        code_pbool + 88,
    ]
    pat_idx = cl(node_patom_token, 1000)
    idx_list.append(pat_idx)
    idx = jnp.stack(idx_list + [jnp.zeros((n,), jnp.int32)] * 6, axis=0)
    idx = idx.reshape(16, nb, BLOCK).transpose(1, 0, 2)

    g = _sc_gather(pt128, pat_idx[:SC_N].reshape(1, SC_N))

    fmapt = jnp.zeros((SMALL_ROWS, 16), jnp.float32)
    for k, (sz, off) in enumerate(zip(sizes, offs)):
        fmapt = fmapt.at[off:off + sz, k].set(1.0)
    fmapt = fmapt.astype(jnp.bfloat16)

    out = pl.pallas_call(
        _fwd_kernel,
        grid=(nb,),
        in_specs=[
            pl.BlockSpec((1, 16, BLOCK), lambda i: (i, 0, 0)),
            pl.BlockSpec((SMALL_ROWS, 16), lambda i: (0, 0)),
            pl.BlockSpec((SMALL_ROWS, 256), lambda i: (0, 0)),
            pl.BlockSpec((BLOCK, 128),
                         lambda i: (jnp.minimum(i, NSC_BLOCKS - 1), 0)),
            pl.BlockSpec((PATOM_ROWS, 32), lambda i: (0, 0)),
            pl.BlockSpec((32, 256), lambda i: (0, 0)),
            pl.BlockSpec((256, 256), lambda i: (0, 0)),
            pl.BlockSpec((1, 256), lambda i: (0, 0)),
        ],
        out_specs=pl.BlockSpec((BLOCK, 256), lambda i: (i, 0)),
        out_shape=jax.ShapeDtypeStruct((n, 256), f32),
        scratch_shapes=[pltpu.VMEM((BLOCK, 256), jnp.float32)],
    )(idx, fmapt, small, g, pt, wp, W2.astype(jnp.bfloat16), b2[None, :])
    return out


# speedup vs baseline: 1.6776x; 1.0341x over previous
import functools

import jax
import jax.numpy as jnp
from jax.experimental import pallas as pl
from jax.experimental.pallas import tpu as pltpu
from jax.experimental.pallas import tpu_sc as plsc

N_TOTAL = 100000
BLOCK = 4000
SMALL_ROWS = 128
PATOM_ROWS = 1024
SC_WINDOW = 128
SC_N = 80000
NSC_BLOCKS = SC_N // BLOCK


def _sc_gather(table, flat_idx):
    n = flat_idx.shape[1]
    mesh = plsc.VectorSubcoreMesh(core_axis_name="core",
                                  subcore_axis_name="subcore")

    @pl.kernel(out_type=jax.ShapeDtypeStruct((n, table.shape[1]),
                                             table.dtype),
               mesh=mesh)
    def k(t_hbm, i_hbm, o_hbm):
        def body(i_vmem, o_vmem):
            pltpu.sync_copy(t_hbm.at[i_vmem.at[0]], o_vmem)

        pltpu.emit_pipeline(
            body,
            grid=(n // SC_WINDOW,),
            in_specs=[pl.BlockSpec((1, SC_WINDOW), lambda i: (0, i))],
            out_specs=[pl.BlockSpec((SC_WINDOW, table.shape[1]),
                                    lambda i: (i, 0))],
            core_axis_name=("core", "subcore"),
            dimension_semantics=(pltpu.PARALLEL,),
        )(i_hbm, o_hbm)

    return k(table, flat_idx)


def _fwd_kernel(idx_ref, fmapt_ref, s_ref, g_ref, pt_ref, wp_ref, w2_ref,
                b2_ref, o_ref, acc_ref):
    bf16 = jnp.bfloat16
    i = pl.program_id(0)
    idx = idx_ref[0]
    idx_bf = idx.astype(bf16)

    sel_t = jax.lax.dot_general(
        fmapt_ref[...], idx_bf, (((1,), (0,)), ((), ())),
        preferred_element_type=jnp.float32)
    row_s = jax.lax.broadcasted_iota(jnp.int32, (SMALL_ROWS, 1), 0)
    mh_t = (sel_t == row_s.astype(jnp.float32)).astype(bf16)

    acc_ref[...] = jax.lax.dot_general(
        mh_t, s_ref[...], (((0,), (0,)), ((), ())),
        preferred_element_type=jnp.float32)

    @pl.when(i < NSC_BLOCKS)
    def _sc_part():
        g = g_ref[:, :32].astype(bf16)
        acc_ref[...] += jnp.dot(g, wp_ref[...],
                                preferred_element_type=jnp.float32)

    @pl.when(i >= NSC_BLOCKS)
    def _tc_part():
        col_p = jax.lax.broadcasted_iota(jnp.int32, (1, PATOM_ROWS), 1)
        oh = (idx[9][:, None] == col_p).astype(bf16)
        gg = jnp.dot(oh, pt_ref[...], preferred_element_type=jnp.float32)
        acc_ref[...] += jnp.dot(gg.astype(bf16), wp_ref[...],
                                preferred_element_type=jnp.float32)

    acc = acc_ref[...]
    act = acc * jax.nn.sigmoid(acc)
    o_ref[...] = (
        jnp.dot(act.astype(bf16), w2_ref[...],
                preferred_element_type=jnp.float32)
        + b2_ref[...]
    )


@functools.partial(jax.jit, static_argnames=())
def kernel(node_element, node_charge, node_aromatic, node_hybridization,
           node_num_rings, node_is_donor, node_is_acceptor, node_is_positive,
           node_is_negative, node_is_hydrophobe, node_patom_token, node_type,
           node_pres_residue_type, node_patom_is_donor, node_patom_is_acceptor,
           node_patom_is_positive, node_patom_is_negative,
           node_patom_is_hydrophobic, elem_table, charge_W, charge_b,
           aromatic_table, hybrid_table, rings_table, bool_W, bool_b,
           patom_table, type_table, res_table, patom_bool_W, patom_bool_b,
           W1, b1, W2, b2):
    n = node_element.shape[0]
    nb = n // BLOCK

    f32 = jnp.float32

    p_elem = elem_table @ W1[0:32]
    p_charge = (jnp.arange(3, dtype=f32)[:, None] @ charge_W
                + charge_b[None, :]) @ W1[32:40]
    p_arom = aromatic_table @ W1[40:48]
    p_hyb = hybrid_table @ W1[48:64]
    p_rings = rings_table @ W1[64:72]
    bits = ((jnp.arange(32)[:, None] >> jnp.arange(5)[None, :]) & 1).astype(f32)
    p_bool = (bits @ bool_W + bool_b[None, :]) @ W1[72:88]
    p_type = type_table @ W1[120:136]
    p_res = res_table @ W1[136:152]
    p_pbool = (bits @ patom_bool_W + patom_bool_b[None, :]) @ W1[152:168]

    p_elem = p_elem + b1[None, :]

    small = jnp.concatenate(
        [p_elem, p_charge, p_arom, p_hyb, p_rings, p_bool, p_type, p_res,
         p_pbool], axis=0)
    small = jnp.pad(small, ((0, SMALL_ROWS - small.shape[0]), (0, 0)))
    small = small.astype(jnp.bfloat16)

    wp = W1[88:120].astype(jnp.bfloat16)
    pt128 = jnp.pad(patom_table, ((0, 0), (0, 96)))
    pt = jnp.pad(patom_table, ((0, PATOM_ROWS - patom_table.shape[0]), (0, 0)))
    pt = pt.astype(jnp.bfloat16)

    def cl(x, r):
        return jnp.clip(x, 0, r - 1).astype(jnp.int32)

    code_bool = (cl(node_is_donor, 2) + 2 * cl(node_is_acceptor, 2)
                 + 4 * cl(node_is_positive, 2) + 8 * cl(node_is_negative, 2)
                 + 16 * cl(node_is_hydrophobe, 2))
    code_pbool = (cl(node_patom_is_donor, 2) + 2 * cl(node_patom_is_acceptor, 2)
                  + 4 * cl(node_patom_is_positive, 2)
                  + 8 * cl(node_patom_is_negative, 2)
                  + 16 * cl(node_patom_is_hydrophobic, 2))

    sizes = [13, 3, 2, 7, 5, 32, 4, 22, 32]
    offs = [0, 13, 16, 18, 25, 30, 62, 66, 88]
    idx_list = [
        cl(node_element, 13) + 0,
        cl(node_charge, 3) + 13,
        cl(node_aromatic, 2) + 16,
        cl(node_hybridization, 7) + 18,
        cl(node_num_rings, 5) + 25,
        code_bool + 30,
        cl(node_type, 4) + 62,
        cl(node_pres_residue_type, 22) + 66,
        code_pbool + 88,
    ]
    pat_idx = cl(node_patom_token, 1000)
    idx_list.append(pat_idx)
    idx = jnp.stack(idx_list + [jnp.zeros((n,), jnp.int32)] * 6, axis=0)
    idx = idx.reshape(16, nb, BLOCK).transpose(1, 0, 2)

    g = _sc_gather(pt128, pat_idx[:SC_N].reshape(1, SC_N))

    fmapt = jnp.zeros((SMALL_ROWS, 16), jnp.float32)
    for k, (sz, off) in enumerate(zip(sizes, offs)):
        fmapt = fmapt.at[off:off + sz, k].set(1.0)
    fmapt = fmapt.astype(jnp.bfloat16)

    out = pl.pallas_call(
        _fwd_kernel,
        grid=(nb,),
        in_specs=[
            pl.BlockSpec((1, 16, BLOCK), lambda i: (i, 0, 0)),
            pl.BlockSpec((SMALL_ROWS, 16), lambda i: (0, 0)),
            pl.BlockSpec((SMALL_ROWS, 256), lambda i: (0, 0)),
            pl.BlockSpec((BLOCK, 128),
                         lambda i: (jnp.minimum(i, NSC_BLOCKS - 1), 0)),
            pl.BlockSpec((PATOM_ROWS, 32), lambda i: (0, 0)),
            pl.BlockSpec((32, 256), lambda i: (0, 0)),
            pl.BlockSpec((256, 256), lambda i: (0, 0)),
            pl.BlockSpec((1, 256), lambda i: (0, 0)),
        ],
        out_specs=pl.BlockSpec((BLOCK, 256), lambda i: (i, 0)),
        out_shape=jax.ShapeDtypeStruct((n, 256), f32),
        scratch_shapes=[pltpu.VMEM((BLOCK, 256), jnp.float32)],
    )(idx, fmapt, small, g, pt, wp, W2.astype(jnp.bfloat16), b2[None, :])
    return out
